# baseline (speedup 1.0000x reference)
import functools

import jax
import jax.numpy as jnp
from jax import lax
from jax.experimental import pallas as pl
from jax.experimental.pallas import tpu as pltpu
from jax.experimental.pallas import tpu_sc as plsc

N = 10000
E = 160000
D0 = 256
HID = 256
EMB = 128
NCLS = 40

NC = 2
NS = 16
LANES = 16
CHUNK = 128
NW = NC * NS

EPAD = ((E + NW * CHUNK - 1) // (NW * CHUNK)) * (NW * CHUNK)
TRASH = N
RPS = 632
DW = 128
NP = RPS * NS

F32 = jnp.float32
BF16 = jnp.bfloat16


def _deg_body(dst_hbm, deg0_hbm, deg1_hbm, ones_v, zer_v, idx_v, shared, sem):
    del sem
    cid = lax.axis_index("c")
    sid = lax.axis_index("s")

    def fill_ones(i, c):
        for f in range(DW // LANES):
            ones_v[i, pl.ds(f * LANES, LANES)] = jnp.full((LANES,), 1.0, F32)
        return c
    lax.fori_loop(0, CHUNK, fill_ones, 0)

    def fill_zeros(i, c):
        for f in range(DW // LANES):
            zer_v[i, pl.ds(f * LANES, LANES)] = jnp.zeros((LANES,), F32)
        return c
    lax.fori_loop(0, 8, fill_zeros, 0)

    def zero_slab(i, c):
        pltpu.sync_copy(zer_v, shared.at[pl.ds(sid * RPS + i * 8, 8)])
        return c
    lax.fori_loop(0, RPS // 8, zero_slab, 0)
    plsc.subcore_barrier()

    wid = sid * NC + cid
    nchunk = EPAD // (NW * CHUNK)
    base = wid * nchunk
    pltpu.sync_copy(dst_hbm.at[pl.ds(base, nchunk)], idx_v)

    def chunk(j, c):
        pltpu.sync_copy(ones_v, shared.at[idx_v.at[j]], add=True)
        return c
    lax.fori_loop(0, nchunk, chunk, 0)
    plsc.subcore_barrier()

    @pl.when(cid == 0)
    def _():
        pltpu.sync_copy(shared.at[pl.ds(sid * RPS, RPS)],
                        deg0_hbm.at[pl.ds(sid * RPS, RPS)])

    @pl.when(cid == 1)
    def _():
        pltpu.sync_copy(shared.at[pl.ds(sid * RPS, RPS)],
                        deg1_hbm.at[pl.ds(sid * RPS, RPS)])


HALF = 2


def _gs_pipeline(g_hbm, shared, idx_s, idx_d, r0, r1, sem0, sem1,
                 sem2, sem3, nstage):
    def pair(jj, c):
        c0, c1 = 2 * jj, 2 * jj + 1

        @pl.when(jj > 0)
        def _():
            pltpu.make_async_copy(r0, shared.at[idx_d.at[c0]], sem2).wait()
            pltpu.make_async_copy(r1, shared.at[idx_d.at[c1]], sem3).wait()

        cp0 = pltpu.async_copy(g_hbm.at[idx_s.at[c0]], r0, sem0)
        cp1 = pltpu.async_copy(g_hbm.at[idx_s.at[c1]], r1, sem1)
        cp0.wait()
        pltpu.async_copy(r0, shared.at[idx_d.at[c0]], sem2, add=True)
        cp1.wait()
        pltpu.async_copy(r1, shared.at[idx_d.at[c1]], sem3, add=True)
        return c
    lax.fori_loop(0, nstage // 2, pair, 0)
    last = nstage - 2
    pltpu.make_async_copy(r0, shared.at[idx_d.at[last]], sem2).wait()
    pltpu.make_async_copy(r1, shared.at[idx_d.at[last + 1]], sem3).wait()


def _conv_body(g0_hbm, g1_hbm, src_hbm, dst_hbm, acc0_hbm, acc1_hbm,
               idx_s, idx_d, r0, r1, shared, sem0, sem1, sem2, sem3):
    cid = lax.axis_index("c")
    sid = lax.axis_index("s")
    nchunk = EPAD // (NS * CHUNK)
    nstage = nchunk // HALF
    base = sid * nchunk

    def run(g_hbm, acc_hbm):
        pltpu.sync_copy(g_hbm.at[pl.ds(sid * RPS, RPS)],
                        shared.at[pl.ds(sid * RPS, RPS)])
        plsc.subcore_barrier()

        for h in range(HALF):
            pltpu.sync_copy(src_hbm.at[pl.ds(base + h * nstage, nstage)], idx_s)
            pltpu.sync_copy(dst_hbm.at[pl.ds(base + h * nstage, nstage)], idx_d)
            _gs_pipeline(g_hbm, shared, idx_s, idx_d, r0, r1, sem0, sem1,
                         sem2, sem3, nstage)
        plsc.subcore_barrier()
        pltpu.sync_copy(shared.at[pl.ds(sid * RPS, RPS)],
                        acc_hbm.at[pl.ds(sid * RPS, RPS)])

    @pl.when(cid == 0)
    def _():
        run(g0_hbm, acc0_hbm)

    @pl.when(cid == 1)
    def _():
        run(g1_hbm, acc1_hbm)


def _conv_es_body(g_hbm, z_hbm, src_hbm, dst_hbm, acc0_hbm, acc1_hbm,
                  idx_s, idx_d, r0, r1, shared, sem0, sem1, sem2, sem3):
    cid = lax.axis_index("c")
    sid = lax.axis_index("s")
    wid = sid * NC + cid
    nchunk = EPAD // (NW * CHUNK)
    base = wid * nchunk

    @pl.when(cid == 0)
    def _():
        pltpu.sync_copy(g_hbm.at[pl.ds(sid * RPS, RPS)],
                        shared.at[pl.ds(sid * RPS, RPS)])

    @pl.when(cid == 1)
    def _():
        pltpu.sync_copy(z_hbm.at[pl.ds(sid * RPS, RPS)],
                        shared.at[pl.ds(sid * RPS, RPS)])

    plsc.subcore_barrier()
    pltpu.sync_copy(src_hbm.at[pl.ds(base, nchunk)], idx_s)
    pltpu.sync_copy(dst_hbm.at[pl.ds(base, nchunk)], idx_d)
    _gs_pipeline(g_hbm, shared, idx_s, idx_d, r0, r1, sem0, sem1,
                 sem2, sem3, nchunk)
    plsc.subcore_barrier()

    @pl.when(cid == 0)
    def _():
        pltpu.sync_copy(shared.at[pl.ds(sid * RPS, RPS)],
                        acc0_hbm.at[pl.ds(sid * RPS, RPS)])

    @pl.when(cid == 1)
    def _():
        pltpu.sync_copy(shared.at[pl.ds(sid * RPS, RPS)],
                        acc1_hbm.at[pl.ds(sid * RPS, RPS)])


def _conv_es_call(g, z, src_blk, dst_blk):
    nchunk = EPAD // (NW * CHUNK)
    return pl.kernel(
        _conv_es_body,
        out_type=(jax.ShapeDtypeStruct((NP, EMB), F32),
                  jax.ShapeDtypeStruct((NP, EMB), F32)),
        mesh=_sc_mesh(),
        scratch_types=[
            pltpu.VMEM((nchunk, CHUNK), jnp.int32),
            pltpu.VMEM((nchunk, CHUNK), jnp.int32),
            pltpu.VMEM((CHUNK, EMB), F32),
            pltpu.VMEM((CHUNK, EMB), F32),
            pltpu.VMEM_SHARED((NP, EMB), F32),
            pltpu.SemaphoreType.DMA,
            pltpu.SemaphoreType.DMA,
            pltpu.SemaphoreType.DMA,
            pltpu.SemaphoreType.DMA,
        ],
    )(g, z, src_blk, dst_blk)


def _sc_mesh():
    return plsc.VectorSubcoreMesh(core_axis_name="c", subcore_axis_name="s",
                                  num_cores=NC, num_subcores=NS)


def _deg_call(dst_blk):
    return pl.kernel(
        _deg_body,
        out_type=(jax.ShapeDtypeStruct((NP, DW), F32),
                  jax.ShapeDtypeStruct((NP, DW), F32)),
        mesh=_sc_mesh(),
        scratch_types=[
            pltpu.VMEM((CHUNK, DW), F32),
            pltpu.VMEM((8, DW), F32),
            pltpu.VMEM((EPAD // (NW * CHUNK), CHUNK), jnp.int32),
            pltpu.VMEM_SHARED((NP, DW), F32),
            pltpu.SemaphoreType.DMA,
        ],
    )(dst_blk)


def _conv_call(g0, g1, src_blk, dst_blk, fh=HID // 2):
    nstage = EPAD // (NS * CHUNK) // HALF
    return pl.kernel(
        _conv_body,
        out_type=(jax.ShapeDtypeStruct((NP, fh), F32),
                  jax.ShapeDtypeStruct((NP, fh), F32)),
        mesh=_sc_mesh(),
        scratch_types=[
            pltpu.VMEM((nstage, CHUNK), jnp.int32),
            pltpu.VMEM((nstage, CHUNK), jnp.int32),
            pltpu.VMEM((CHUNK, fh), F32),
            pltpu.VMEM((CHUNK, fh), F32),
            pltpu.VMEM_SHARED((NP, fh), F32),
            pltpu.SemaphoreType.DMA,
            pltpu.SemaphoreType.DMA,
            pltpu.SemaphoreType.DMA,
            pltpu.SemaphoreType.DMA,
        ],
    )(g0, g1, src_blk, dst_blk)


BM_A = 400
BM_E = 632


def _mm_body(x_ref, emb_ref, w1_ref, out_ref):
    xb = x_ref[...].astype(BF16)
    h = jnp.maximum(jnp.dot(xb, emb_ref[...],
                            preferred_element_type=F32), 0.0)
    out_ref[...] = jnp.dot(h, w1_ref[...], preferred_element_type=F32)


def _mm_call(x, emb, w1):
    grid = (N // BM_A,)
    return pl.pallas_call(
        _mm_body,
        grid=grid,
        in_specs=[
            pl.BlockSpec((BM_A, N), lambda m: (m, 0)),
            pl.BlockSpec((N, D0), lambda m: (0, 0)),
            pl.BlockSpec((D0, HID), lambda m: (0, 0)),
        ],
        out_specs=pl.BlockSpec((BM_A, HID), lambda m: (m, 0)),
        out_shape=jax.ShapeDtypeStruct((N, HID), F32),
        compiler_params=pltpu.CompilerParams(
            dimension_semantics=("arbitrary",)),
    )(x, emb, w1)


def _dinv(d0_ref, d1_ref):
    deg = d0_ref[:, 0:1] + d1_ref[:, 0:1] + 1.0
    return lax.rsqrt(deg)


def _scale_split_body(hw_ref, d0_ref, d1_ref, g0_ref, g1_ref):
    g = hw_ref[...] * _dinv(d0_ref, d1_ref)
    g0_ref[...] = g[:, :HID // 2]
    g1_ref[...] = g[:, HID // 2:]


def _scale_split_call(hw1, d0, d1):
    grid = (NP // BM_E,)
    return pl.pallas_call(
        _scale_split_body,
        grid=grid,
        in_specs=[
            pl.BlockSpec((BM_E, HID), lambda m: (m, 0)),
            pl.BlockSpec((BM_E, DW), lambda m: (m, 0)),
            pl.BlockSpec((BM_E, DW), lambda m: (m, 0)),
        ],
        out_specs=(pl.BlockSpec((BM_E, HID // 2), lambda m: (m, 0)),
                   pl.BlockSpec((BM_E, HID // 2), lambda m: (m, 0))),
        out_shape=(jax.ShapeDtypeStruct((NP, HID // 2), F32),
                   jax.ShapeDtypeStruct((NP, HID // 2), F32)),
    )(hw1, d0, d1)


def _layer2_body(acc_ref, d0_ref, d1_ref, b1_ref, w2_ref, g2_ref):
    dinv = _dinv(d0_ref, d1_ref)
    h2 = jnp.maximum(acc_ref[...] * dinv + b1_ref[...], 0.0)
    g2_ref[...] = jnp.dot(h2, w2_ref[...], preferred_element_type=F32) * dinv


def _layer2_call(acc1, d0, d1, b1, w2):
    grid = (NP // BM_E,)
    return pl.pallas_call(
        _layer2_body,
        grid=grid,
        in_specs=[
            pl.BlockSpec((BM_E, HID), lambda m: (m, 0)),
            pl.BlockSpec((BM_E, DW), lambda m: (m, 0)),
            pl.BlockSpec((BM_E, DW), lambda m: (m, 0)),
            pl.BlockSpec((1, HID), lambda m: (0, 0)),
            pl.BlockSpec((HID, EMB), lambda m: (0, 0)),
        ],
        out_specs=pl.BlockSpec((BM_E, EMB), lambda m: (m, 0)),
        out_shape=jax.ShapeDtypeStruct((NP, EMB), F32),
    )(acc1, d0, d1, b1, w2)


def _decoder_body(a0_ref, a1_ref, d0_ref, d1_ref, b2_ref, wd_ref, out_ref):
    dinv = _dinv(d0_ref, d1_ref)
    acc = a0_ref[...] + a1_ref[...]
    h3 = jnp.maximum(acc * dinv + b2_ref[...], 0.0)
    out_ref[...] = jnp.dot(h3, wd_ref[...], preferred_element_type=F32)


def _decoder_call(a0, a1, d0, d1, b2, wd):
    grid = (NP // BM_E,)
    return pl.pallas_call(
        _decoder_body,
        grid=grid,
        in_specs=[
            pl.BlockSpec((BM_E, EMB), lambda m: (m, 0)),
            pl.BlockSpec((BM_E, EMB), lambda m: (m, 0)),
            pl.BlockSpec((BM_E, DW), lambda m: (m, 0)),
            pl.BlockSpec((BM_E, DW), lambda m: (m, 0)),
            pl.BlockSpec((1, EMB), lambda m: (0, 0)),
            pl.BlockSpec((EMB, NCLS), lambda m: (0, 0)),
        ],
        out_specs=pl.BlockSpec((BM_E, NCLS), lambda m: (m, 0)),
        out_shape=jax.ShapeDtypeStruct((N, NCLS), F32),
    )(a0, a1, d0, d1, b2, wd)


def kernel(x, edge_index, embedding, W1, b1, W2, b2, Wdec):
    pad = EPAD - E
    src_p = jnp.concatenate([edge_index[0],
                             jnp.zeros((pad,), edge_index.dtype)])
    dst_p = jnp.concatenate([edge_index[1],
                             jnp.full((pad,), TRASH, edge_index.dtype)])
    src_blk = src_p.reshape(-1, CHUNK)
    dst_blk = dst_p.reshape(-1, CHUNK)

    d0, d1 = _deg_call(dst_blk)

    hw1 = _mm_call(x, embedding.astype(BF16), W1)
    g1a, g1b = _scale_split_call(hw1, d0, d1)
    acc1a, acc1b = _conv_call(g1a, g1b, src_blk, dst_blk)
    acc1 = jnp.concatenate([acc1a, acc1b], axis=1)

    g2 = _layer2_call(acc1, d0, d1, b1.reshape(1, -1), W2)
    z = jnp.zeros((NP, EMB), F32)
    p0, p1 = _conv_es_call(g2, z, src_blk, dst_blk)

    return _decoder_call(p0, p1, d0, d1, b2.reshape(1, -1), Wdec)


# speedup vs baseline: 1.1283x; 1.1283x over previous
import functools

import jax
import jax.numpy as jnp
from jax import lax
from jax.experimental import pallas as pl
from jax.experimental.pallas import tpu as pltpu
from jax.experimental.pallas import tpu_sc as plsc

N = 10000
E = 160000
D0 = 256
HID = 256
EMB = 128
NCLS = 40

NC = 2
NS = 16
LANES = 16
CHUNK = 128
NW = NC * NS

EPAD = ((E + NW * CHUNK - 1) // (NW * CHUNK)) * (NW * CHUNK)
TRASH = N
RPS = 632
DW = 128
NP = RPS * NS

F32 = jnp.float32
BF16 = jnp.bfloat16


def _deg_body(dst_hbm, deg0_hbm, deg1_hbm, ones_v, zer_v, idx_v, shared, sem):
    del sem
    cid = lax.axis_index("c")
    sid = lax.axis_index("s")

    def fill_ones(i, c):
        for f in range(DW // LANES):
            ones_v[i, pl.ds(f * LANES, LANES)] = jnp.full((LANES,), 1.0, F32)
        return c
    lax.fori_loop(0, CHUNK, fill_ones, 0)

    def fill_zeros(i, c):
        for f in range(DW // LANES):
            zer_v[i, pl.ds(f * LANES, LANES)] = jnp.zeros((LANES,), F32)
        return c
    lax.fori_loop(0, 8, fill_zeros, 0)

    def zero_slab(i, c):
        pltpu.sync_copy(zer_v, shared.at[pl.ds(sid * RPS + i * 8, 8)])
        return c
    lax.fori_loop(0, RPS // 8, zero_slab, 0)
    plsc.subcore_barrier()

    wid = sid * NC + cid
    nchunk = EPAD // (NW * CHUNK)
    base = wid * nchunk
    pltpu.sync_copy(dst_hbm.at[pl.ds(base, nchunk)], idx_v)

    def chunk(j, c):
        pltpu.sync_copy(ones_v, shared.at[idx_v.at[j]], add=True)
        return c
    lax.fori_loop(0, nchunk, chunk, 0)
    plsc.subcore_barrier()

    @pl.when(cid == 0)
    def _():
        pltpu.sync_copy(shared.at[pl.ds(sid * RPS, RPS)],
                        deg0_hbm.at[pl.ds(sid * RPS, RPS)])

    @pl.when(cid == 1)
    def _():
        pltpu.sync_copy(shared.at[pl.ds(sid * RPS, RPS)],
                        deg1_hbm.at[pl.ds(sid * RPS, RPS)])


HALF = 2


def _gs_pipeline(g_hbm, shared, idx_s, idx_d, r0, r1, sem0, sem1,
                 sem2, sem3, nstage):
    def pair(jj, c):
        c0, c1 = 2 * jj, 2 * jj + 1

        @pl.when(jj > 0)
        def _():
            pltpu.make_async_copy(r0, shared.at[idx_d.at[c0]], sem2).wait()
            pltpu.make_async_copy(r1, shared.at[idx_d.at[c1]], sem3).wait()

        cp0 = pltpu.async_copy(g_hbm.at[idx_s.at[c0]], r0, sem0)
        cp1 = pltpu.async_copy(g_hbm.at[idx_s.at[c1]], r1, sem1)
        cp0.wait()
        pltpu.async_copy(r0, shared.at[idx_d.at[c0]], sem2, add=True)
        cp1.wait()
        pltpu.async_copy(r1, shared.at[idx_d.at[c1]], sem3, add=True)
        return c
    lax.fori_loop(0, nstage // 2, pair, 0)
    last = nstage - 2
    pltpu.make_async_copy(r0, shared.at[idx_d.at[last]], sem2).wait()
    pltpu.make_async_copy(r1, shared.at[idx_d.at[last + 1]], sem3).wait()


def _conv_body(g0_hbm, g1_hbm, src_hbm, dst_hbm, acc0_hbm, acc1_hbm,
               idx_s, idx_d, r0, r1, shared, sem0, sem1, sem2, sem3):
    cid = lax.axis_index("c")
    sid = lax.axis_index("s")
    nchunk = EPAD // (NS * CHUNK)
    nstage = nchunk // HALF
    base = sid * nchunk

    def run(g_hbm, acc_hbm):
        pltpu.sync_copy(g_hbm.at[pl.ds(sid * RPS, RPS)],
                        shared.at[pl.ds(sid * RPS, RPS)])
        plsc.subcore_barrier()

        for h in range(HALF):
            pltpu.sync_copy(src_hbm.at[pl.ds(base + h * nstage, nstage)], idx_s)
            pltpu.sync_copy(dst_hbm.at[pl.ds(base + h * nstage, nstage)], idx_d)
            _gs_pipeline(g_hbm, shared, idx_s, idx_d, r0, r1, sem0, sem1,
                         sem2, sem3, nstage)
        plsc.subcore_barrier()
        pltpu.sync_copy(shared.at[pl.ds(sid * RPS, RPS)],
                        acc_hbm.at[pl.ds(sid * RPS, RPS)])

    @pl.when(cid == 0)
    def _():
        run(g0_hbm, acc0_hbm)

    @pl.when(cid == 1)
    def _():
        run(g1_hbm, acc1_hbm)


def _conv_es_body(g_hbm, z_hbm, src_hbm, dst_hbm, acc0_hbm, acc1_hbm,
                  idx_s, idx_d, r0, r1, shared, sem0, sem1, sem2, sem3):
    cid = lax.axis_index("c")
    sid = lax.axis_index("s")
    wid = sid * NC + cid
    nchunk = EPAD // (NW * CHUNK)
    base = wid * nchunk

    @pl.when(cid == 0)
    def _():
        pltpu.sync_copy(g_hbm.at[pl.ds(sid * RPS, RPS)],
                        shared.at[pl.ds(sid * RPS, RPS)])

    @pl.when(cid == 1)
    def _():
        pltpu.sync_copy(z_hbm.at[pl.ds(sid * RPS, RPS)],
                        shared.at[pl.ds(sid * RPS, RPS)])

    plsc.subcore_barrier()
    pltpu.sync_copy(src_hbm.at[pl.ds(base, nchunk)], idx_s)
    pltpu.sync_copy(dst_hbm.at[pl.ds(base, nchunk)], idx_d)
    _gs_pipeline(g_hbm, shared, idx_s, idx_d, r0, r1, sem0, sem1,
                 sem2, sem3, nchunk)
    plsc.subcore_barrier()

    @pl.when(cid == 0)
    def _():
        pltpu.sync_copy(shared.at[pl.ds(sid * RPS, RPS)],
                        acc0_hbm.at[pl.ds(sid * RPS, RPS)])

    @pl.when(cid == 1)
    def _():
        pltpu.sync_copy(shared.at[pl.ds(sid * RPS, RPS)],
                        acc1_hbm.at[pl.ds(sid * RPS, RPS)])


def _conv_es_call(g, z, src_blk, dst_blk):
    nchunk = EPAD // (NW * CHUNK)
    return pl.kernel(
        _conv_es_body,
        out_type=(jax.ShapeDtypeStruct((NP, EMB), F32),
                  jax.ShapeDtypeStruct((NP, EMB), F32)),
        mesh=_sc_mesh(),
        scratch_types=[
            pltpu.VMEM((nchunk, CHUNK), jnp.int32),
            pltpu.VMEM((nchunk, CHUNK), jnp.int32),
            pltpu.VMEM((CHUNK, EMB), F32),
            pltpu.VMEM((CHUNK, EMB), F32),
            pltpu.VMEM_SHARED((NP, EMB), F32),
            pltpu.SemaphoreType.DMA,
            pltpu.SemaphoreType.DMA,
            pltpu.SemaphoreType.DMA,
            pltpu.SemaphoreType.DMA,
        ],
    )(g, z, src_blk, dst_blk)


def _sc_mesh():
    return plsc.VectorSubcoreMesh(core_axis_name="c", subcore_axis_name="s",
                                  num_cores=NC, num_subcores=NS)


def _deg_call(dst_blk):
    return pl.kernel(
        _deg_body,
        out_type=(jax.ShapeDtypeStruct((NP, DW), F32),
                  jax.ShapeDtypeStruct((NP, DW), F32)),
        mesh=_sc_mesh(),
        scratch_types=[
            pltpu.VMEM((CHUNK, DW), F32),
            pltpu.VMEM((8, DW), F32),
            pltpu.VMEM((EPAD // (NW * CHUNK), CHUNK), jnp.int32),
            pltpu.VMEM_SHARED((NP, DW), F32),
            pltpu.SemaphoreType.DMA,
        ],
    )(dst_blk)


def _conv_call(g0, g1, src_blk, dst_blk, fh=HID // 2):
    nstage = EPAD // (NS * CHUNK) // HALF
    return pl.kernel(
        _conv_body,
        out_type=(jax.ShapeDtypeStruct((NP, fh), F32),
                  jax.ShapeDtypeStruct((NP, fh), F32)),
        mesh=_sc_mesh(),
        scratch_types=[
            pltpu.VMEM((nstage, CHUNK), jnp.int32),
            pltpu.VMEM((nstage, CHUNK), jnp.int32),
            pltpu.VMEM((CHUNK, fh), F32),
            pltpu.VMEM((CHUNK, fh), F32),
            pltpu.VMEM_SHARED((NP, fh), F32),
            pltpu.SemaphoreType.DMA,
            pltpu.SemaphoreType.DMA,
            pltpu.SemaphoreType.DMA,
            pltpu.SemaphoreType.DMA,
        ],
    )(g0, g1, src_blk, dst_blk)


BM_A = 400
BM_E = 632


def _mm_body(x_ref, emb_ref, w1_ref, d0_ref, d1_ref, g0_ref, g1_ref):
    xb = x_ref[...].astype(BF16)
    h = jnp.maximum(jnp.dot(xb, emb_ref[...],
                            preferred_element_type=F32), 0.0)
    g = jnp.dot(h, w1_ref[...],
                preferred_element_type=F32) * _dinv(d0_ref, d1_ref)
    g0_ref[...] = g[:, :HID // 2]
    g1_ref[...] = g[:, HID // 2:]


def _mm_call(x, emb, w1, d0, d1):
    grid = (N // BM_A,)
    return pl.pallas_call(
        _mm_body,
        grid=grid,
        in_specs=[
            pl.BlockSpec((BM_A, N), lambda m: (m, 0)),
            pl.BlockSpec((N, D0), lambda m: (0, 0)),
            pl.BlockSpec((D0, HID), lambda m: (0, 0)),
            pl.BlockSpec((BM_A, DW), lambda m: (m, 0)),
            pl.BlockSpec((BM_A, DW), lambda m: (m, 0)),
        ],
        out_specs=(pl.BlockSpec((BM_A, HID // 2), lambda m: (m, 0)),
                   pl.BlockSpec((BM_A, HID // 2), lambda m: (m, 0))),
        out_shape=(jax.ShapeDtypeStruct((NP, HID // 2), F32),
                   jax.ShapeDtypeStruct((NP, HID // 2), F32)),
        compiler_params=pltpu.CompilerParams(
            dimension_semantics=("arbitrary",)),
    )(x, emb, w1, d0, d1)


def _dinv(d0_ref, d1_ref):
    deg = d0_ref[:, 0:1] + d1_ref[:, 0:1] + 1.0
    return lax.rsqrt(deg)


def _scale_split_body(hw_ref, d0_ref, d1_ref, g0_ref, g1_ref):
    g = hw_ref[...] * _dinv(d0_ref, d1_ref)
    g0_ref[...] = g[:, :HID // 2]
    g1_ref[...] = g[:, HID // 2:]


def _scale_split_call(hw1, d0, d1):
    grid = (NP // BM_E,)
    return pl.pallas_call(
        _scale_split_body,
        grid=grid,
        in_specs=[
            pl.BlockSpec((BM_E, HID), lambda m: (m, 0)),
            pl.BlockSpec((BM_E, DW), lambda m: (m, 0)),
            pl.BlockSpec((BM_E, DW), lambda m: (m, 0)),
        ],
        out_specs=(pl.BlockSpec((BM_E, HID // 2), lambda m: (m, 0)),
                   pl.BlockSpec((BM_E, HID // 2), lambda m: (m, 0))),
        out_shape=(jax.ShapeDtypeStruct((NP, HID // 2), F32),
                   jax.ShapeDtypeStruct((NP, HID // 2), F32)),
    )(hw1, d0, d1)


def _layer2_body(acc_ref, d0_ref, d1_ref, b1_ref, w2_ref, g2_ref):
    dinv = _dinv(d0_ref, d1_ref)
    h2 = jnp.maximum(acc_ref[...] * dinv + b1_ref[...], 0.0)
    g2_ref[...] = jnp.dot(h2, w2_ref[...], preferred_element_type=F32) * dinv


def _layer2_call(acc1, d0, d1, b1, w2):
    grid = (NP // BM_E,)
    return pl.pallas_call(
        _layer2_body,
        grid=grid,
        in_specs=[
            pl.BlockSpec((BM_E, HID), lambda m: (m, 0)),
            pl.BlockSpec((BM_E, DW), lambda m: (m, 0)),
            pl.BlockSpec((BM_E, DW), lambda m: (m, 0)),
            pl.BlockSpec((1, HID), lambda m: (0, 0)),
            pl.BlockSpec((HID, EMB), lambda m: (0, 0)),
        ],
        out_specs=pl.BlockSpec((BM_E, EMB), lambda m: (m, 0)),
        out_shape=jax.ShapeDtypeStruct((NP, EMB), F32),
    )(acc1, d0, d1, b1, w2)


def _decoder_body(a0_ref, a1_ref, d0_ref, d1_ref, b2_ref, wd_ref, out_ref):
    dinv = _dinv(d0_ref, d1_ref)
    acc = a0_ref[...] + a1_ref[...]
    h3 = jnp.maximum(acc * dinv + b2_ref[...], 0.0)
    out_ref[...] = jnp.dot(h3, wd_ref[...], preferred_element_type=F32)


def _decoder_call(a0, a1, d0, d1, b2, wd):
    grid = (NP // BM_E,)
    return pl.pallas_call(
        _decoder_body,
        grid=grid,
        in_specs=[
            pl.BlockSpec((BM_E, EMB), lambda m: (m, 0)),
            pl.BlockSpec((BM_E, EMB), lambda m: (m, 0)),
            pl.BlockSpec((BM_E, DW), lambda m: (m, 0)),
            pl.BlockSpec((BM_E, DW), lambda m: (m, 0)),
            pl.BlockSpec((1, EMB), lambda m: (0, 0)),
            pl.BlockSpec((EMB, NCLS), lambda m: (0, 0)),
        ],
        out_specs=pl.BlockSpec((BM_E, NCLS), lambda m: (m, 0)),
        out_shape=jax.ShapeDtypeStruct((N, NCLS), F32),
    )(a0, a1, d0, d1, b2, wd)


def kernel(x, edge_index, embedding, W1, b1, W2, b2, Wdec):
    pad = EPAD - E
    src_p = jnp.concatenate([edge_index[0],
                             jnp.zeros((pad,), edge_index.dtype)])
    dst_p = jnp.concatenate([edge_index[1],
                             jnp.full((pad,), TRASH, edge_index.dtype)])
    src_blk = src_p.reshape(-1, CHUNK)
    dst_blk = dst_p.reshape(-1, CHUNK)

    d0, d1 = _deg_call(dst_blk)

    g1a, g1b = _mm_call(x, embedding.astype(BF16), W1, d0, d1)
    acc1a, acc1b = _conv_call(g1a, g1b, src_blk, dst_blk)
    acc1 = jnp.concatenate([acc1a, acc1b], axis=1)

    g2 = _layer2_call(acc1, d0, d1, b1.reshape(1, -1), W2)
    z = jnp.zeros((NP, EMB), F32)
    p0, p1 = _conv_es_call(g2, z, src_blk, dst_blk)

    return _decoder_call(p0, p1, d0, d1, b2.reshape(1, -1), Wdec)


# speedup vs baseline: 1.1294x; 1.0010x over previous
import jax
import jax.numpy as jnp
from jax import lax
from jax.experimental import pallas as pl
from jax.experimental.pallas import tpu as pltpu
from jax.experimental.pallas import tpu_sc as plsc

N = 10000
E = 160000
D0 = 256
HID = 256
EMB = 128
NCLS = 40

NC = 2
NS = 16
LANES = 16
CHUNK = 128
NW = NC * NS

EPAD = ((E + NW * CHUNK - 1) // (NW * CHUNK)) * (NW * CHUNK)
TRASH = N
RPS = 632
DW = 128
NP = RPS * NS

F32 = jnp.float32
BF16 = jnp.bfloat16


def _deg_body(dst_hbm, deg0_hbm, deg1_hbm, ones_v, zer_v, idx_v, shared, sem):
    del sem
    cid = lax.axis_index("c")
    sid = lax.axis_index("s")

    def fill_ones(i, c):
        for f in range(DW // LANES):
            ones_v[i, pl.ds(f * LANES, LANES)] = jnp.full((LANES,), 1.0, F32)
        return c
    lax.fori_loop(0, CHUNK, fill_ones, 0)

    def fill_zeros(i, c):
        for f in range(DW // LANES):
            zer_v[i, pl.ds(f * LANES, LANES)] = jnp.zeros((LANES,), F32)
        return c
    lax.fori_loop(0, 8, fill_zeros, 0)

    def zero_slab(i, c):
        pltpu.sync_copy(zer_v, shared.at[pl.ds(sid * RPS + i * 8, 8)])
        return c
    lax.fori_loop(0, RPS // 8, zero_slab, 0)
    plsc.subcore_barrier()

    wid = sid * NC + cid
    nchunk = EPAD // (NW * CHUNK)
    base = wid * nchunk
    pltpu.sync_copy(dst_hbm.at[pl.ds(base, nchunk)], idx_v)

    def chunk(j, c):
        pltpu.sync_copy(ones_v, shared.at[idx_v.at[j]], add=True)
        return c
    lax.fori_loop(0, nchunk, chunk, 0)
    plsc.subcore_barrier()

    @pl.when(cid == 0)
    def _():
        pltpu.sync_copy(shared.at[pl.ds(sid * RPS, RPS)],
                        deg0_hbm.at[pl.ds(sid * RPS, RPS)])

    @pl.when(cid == 1)
    def _():
        pltpu.sync_copy(shared.at[pl.ds(sid * RPS, RPS)],
                        deg1_hbm.at[pl.ds(sid * RPS, RPS)])


HALF = 2


def _gs_pipeline(g_hbm, shared, idx_s, idx_d, r0, r1, sem0, sem1,
                 sem2, sem3, nstage):
    def pair(jj, c):
        c0, c1 = 2 * jj, 2 * jj + 1

        @pl.when(jj > 0)
        def _():
            pltpu.make_async_copy(r0, shared.at[idx_d.at[c0]], sem2).wait()
            pltpu.make_async_copy(r1, shared.at[idx_d.at[c1]], sem3).wait()

        cp0 = pltpu.async_copy(g_hbm.at[idx_s.at[c0]], r0, sem0)
        cp1 = pltpu.async_copy(g_hbm.at[idx_s.at[c1]], r1, sem1)
        cp0.wait()
        pltpu.async_copy(r0, shared.at[idx_d.at[c0]], sem2, add=True)
        cp1.wait()
        pltpu.async_copy(r1, shared.at[idx_d.at[c1]], sem3, add=True)
        return c
    lax.fori_loop(0, nstage // 2, pair, 0)
    last = nstage - 2
    pltpu.make_async_copy(r0, shared.at[idx_d.at[last]], sem2).wait()
    pltpu.make_async_copy(r1, shared.at[idx_d.at[last + 1]], sem3).wait()


def _conv_body(g0_hbm, g1_hbm, src_hbm, dst_hbm, acc0_hbm, acc1_hbm,
               idx_s, idx_d, r0, r1, shared, sem0, sem1, sem2, sem3):
    cid = lax.axis_index("c")
    sid = lax.axis_index("s")
    nchunk = EPAD // (NS * CHUNK)
    nstage = nchunk // HALF
    base = sid * nchunk

    def run(g_hbm, acc_hbm):
        pltpu.sync_copy(g_hbm.at[pl.ds(sid * RPS, RPS)],
                        shared.at[pl.ds(sid * RPS, RPS)])
        plsc.subcore_barrier()

        for h in range(HALF):
            pltpu.sync_copy(src_hbm.at[pl.ds(base + h * nstage, nstage)], idx_s)
            pltpu.sync_copy(dst_hbm.at[pl.ds(base + h * nstage, nstage)], idx_d)
            _gs_pipeline(g_hbm, shared, idx_s, idx_d, r0, r1, sem0, sem1,
                         sem2, sem3, nstage)
        plsc.subcore_barrier()
        pltpu.sync_copy(shared.at[pl.ds(sid * RPS, RPS)],
                        acc_hbm.at[pl.ds(sid * RPS, RPS)])

    @pl.when(cid == 0)
    def _():
        run(g0_hbm, acc0_hbm)

    @pl.when(cid == 1)
    def _():
        run(g1_hbm, acc1_hbm)


def _conv_es_body(g_hbm, z_hbm, src_hbm, dst_hbm, acc0_hbm, acc1_hbm,
                  idx_s, idx_d, r0, r1, shared, sem0, sem1, sem2, sem3):
    cid = lax.axis_index("c")
    sid = lax.axis_index("s")
    wid = sid * NC + cid
    nchunk = EPAD // (NW * CHUNK)
    base = wid * nchunk

    @pl.when(cid == 0)
    def _():
        pltpu.sync_copy(g_hbm.at[pl.ds(sid * RPS, RPS)],
                        shared.at[pl.ds(sid * RPS, RPS)])

    @pl.when(cid == 1)
    def _():
        pltpu.sync_copy(z_hbm.at[pl.ds(sid * RPS, RPS)],
                        shared.at[pl.ds(sid * RPS, RPS)])

    plsc.subcore_barrier()
    pltpu.sync_copy(src_hbm.at[pl.ds(base, nchunk)], idx_s)
    pltpu.sync_copy(dst_hbm.at[pl.ds(base, nchunk)], idx_d)
    _gs_pipeline(g_hbm, shared, idx_s, idx_d, r0, r1, sem0, sem1,
                 sem2, sem3, nchunk)
    plsc.subcore_barrier()

    @pl.when(cid == 0)
    def _():
        pltpu.sync_copy(shared.at[pl.ds(sid * RPS, RPS)],
                        acc0_hbm.at[pl.ds(sid * RPS, RPS)])

    @pl.when(cid == 1)
    def _():
        pltpu.sync_copy(shared.at[pl.ds(sid * RPS, RPS)],
                        acc1_hbm.at[pl.ds(sid * RPS, RPS)])


def _conv_es_call(g, z, src_blk, dst_blk):
    nchunk = EPAD // (NW * CHUNK)
    return pl.kernel(
        _conv_es_body,
        out_type=(jax.ShapeDtypeStruct((NP, EMB), F32),
                  jax.ShapeDtypeStruct((NP, EMB), F32)),
        mesh=_sc_mesh(),
        scratch_types=[
            pltpu.VMEM((nchunk, CHUNK), jnp.int32),
            pltpu.VMEM((nchunk, CHUNK), jnp.int32),
            pltpu.VMEM((CHUNK, EMB), F32),
            pltpu.VMEM((CHUNK, EMB), F32),
            pltpu.VMEM_SHARED((NP, EMB), F32),
            pltpu.SemaphoreType.DMA,
            pltpu.SemaphoreType.DMA,
            pltpu.SemaphoreType.DMA,
            pltpu.SemaphoreType.DMA,
        ],
    )(g, z, src_blk, dst_blk)


def _sc_mesh():
    return plsc.VectorSubcoreMesh(core_axis_name="c", subcore_axis_name="s",
                                  num_cores=NC, num_subcores=NS)


def _deg_call(dst_blk):
    return pl.kernel(
        _deg_body,
        out_type=(jax.ShapeDtypeStruct((NP, DW), F32),
                  jax.ShapeDtypeStruct((NP, DW), F32)),
        mesh=_sc_mesh(),
        scratch_types=[
            pltpu.VMEM((CHUNK, DW), F32),
            pltpu.VMEM((8, DW), F32),
            pltpu.VMEM((EPAD // (NW * CHUNK), CHUNK), jnp.int32),
            pltpu.VMEM_SHARED((NP, DW), F32),
            pltpu.SemaphoreType.DMA,
        ],
    )(dst_blk)


def _conv_call(g0, g1, src_blk, dst_blk, fh=HID // 2):
    nstage = EPAD // (NS * CHUNK) // HALF
    return pl.kernel(
        _conv_body,
        out_type=(jax.ShapeDtypeStruct((NP, fh), F32),
                  jax.ShapeDtypeStruct((NP, fh), F32)),
        mesh=_sc_mesh(),
        scratch_types=[
            pltpu.VMEM((nstage, CHUNK), jnp.int32),
            pltpu.VMEM((nstage, CHUNK), jnp.int32),
            pltpu.VMEM((CHUNK, fh), F32),
            pltpu.VMEM((CHUNK, fh), F32),
            pltpu.VMEM_SHARED((NP, fh), F32),
            pltpu.SemaphoreType.DMA,
            pltpu.SemaphoreType.DMA,
            pltpu.SemaphoreType.DMA,
            pltpu.SemaphoreType.DMA,
        ],
    )(g0, g1, src_blk, dst_blk)


BM_A = 400
BM_E = 632


def _mm_body(x_ref, emb_ref, w1_ref, d0_ref, d1_ref, g0_ref, g1_ref):
    xb = x_ref[...].astype(BF16)
    h = jnp.maximum(jnp.dot(xb, emb_ref[...],
                            preferred_element_type=F32), 0.0)
    g = jnp.dot(h, w1_ref[...],
                preferred_element_type=F32) * _dinv(d0_ref, d1_ref)
    g0_ref[...] = g[:, :HID // 2]
    g1_ref[...] = g[:, HID // 2:]


def _mm_call(x, emb, w1, d0, d1):
    grid = (N // BM_A,)
    return pl.pallas_call(
        _mm_body,
        grid=grid,
        in_specs=[
            pl.BlockSpec((BM_A, N), lambda m: (m, 0)),
            pl.BlockSpec((N, D0), lambda m: (0, 0)),
            pl.BlockSpec((D0, HID), lambda m: (0, 0)),
            pl.BlockSpec((BM_A, DW), lambda m: (m, 0)),
            pl.BlockSpec((BM_A, DW), lambda m: (m, 0)),
        ],
        out_specs=(pl.BlockSpec((BM_A, HID // 2), lambda m: (m, 0)),
                   pl.BlockSpec((BM_A, HID // 2), lambda m: (m, 0))),
        out_shape=(jax.ShapeDtypeStruct((NP, HID // 2), F32),
                   jax.ShapeDtypeStruct((NP, HID // 2), F32)),
        compiler_params=pltpu.CompilerParams(
            dimension_semantics=("arbitrary",)),
    )(x, emb, w1, d0, d1)


def _dinv(d0_ref, d1_ref):
    deg = d0_ref[:, 0:1] + d1_ref[:, 0:1] + 1.0
    return lax.rsqrt(deg)


def _layer2_body(acc_ref, d0_ref, d1_ref, b1_ref, w2_ref, g2_ref):
    dinv = _dinv(d0_ref, d1_ref)
    h2 = jnp.maximum(acc_ref[...] * dinv + b1_ref[...], 0.0)
    g2_ref[...] = jnp.dot(h2, w2_ref[...], preferred_element_type=F32) * dinv


def _layer2_call(acc1, d0, d1, b1, w2):
    grid = (NP // BM_E,)
    return pl.pallas_call(
        _layer2_body,
        grid=grid,
        in_specs=[
            pl.BlockSpec((BM_E, HID), lambda m: (m, 0)),
            pl.BlockSpec((BM_E, DW), lambda m: (m, 0)),
            pl.BlockSpec((BM_E, DW), lambda m: (m, 0)),
            pl.BlockSpec((1, HID), lambda m: (0, 0)),
            pl.BlockSpec((HID, EMB), lambda m: (0, 0)),
        ],
        out_specs=pl.BlockSpec((BM_E, EMB), lambda m: (m, 0)),
        out_shape=jax.ShapeDtypeStruct((NP, EMB), F32),
    )(acc1, d0, d1, b1, w2)


def _decoder_body(a0_ref, a1_ref, d0_ref, d1_ref, b2_ref, wd_ref, out_ref):
    dinv = _dinv(d0_ref, d1_ref)
    acc = a0_ref[...] + a1_ref[...]
    h3 = jnp.maximum(acc * dinv + b2_ref[...], 0.0)
    out_ref[...] = jnp.dot(h3, wd_ref[...], preferred_element_type=F32)


def _decoder_call(a0, a1, d0, d1, b2, wd):
    grid = (NP // BM_E,)
    return pl.pallas_call(
        _decoder_body,
        grid=grid,
        in_specs=[
            pl.BlockSpec((BM_E, EMB), lambda m: (m, 0)),
            pl.BlockSpec((BM_E, EMB), lambda m: (m, 0)),
            pl.BlockSpec((BM_E, DW), lambda m: (m, 0)),
            pl.BlockSpec((BM_E, DW), lambda m: (m, 0)),
            pl.BlockSpec((1, EMB), lambda m: (0, 0)),
            pl.BlockSpec((EMB, NCLS), lambda m: (0, 0)),
        ],
        out_specs=pl.BlockSpec((BM_E, NCLS), lambda m: (m, 0)),
        out_shape=jax.ShapeDtypeStruct((N, NCLS), F32),
    )(a0, a1, d0, d1, b2, wd)


def kernel(x, edge_index, embedding, W1, b1, W2, b2, Wdec):
    pad = EPAD - E
    src_p = jnp.concatenate([edge_index[0],
                             jnp.zeros((pad,), edge_index.dtype)])
    dst_p = jnp.concatenate([edge_index[1],
                             jnp.full((pad,), TRASH, edge_index.dtype)])
    src_blk = src_p.reshape(-1, CHUNK)
    dst_blk = dst_p.reshape(-1, CHUNK)

    d0, d1 = _deg_call(dst_blk)

    g1a, g1b = _mm_call(x, embedding.astype(BF16), W1, d0, d1)
    acc1a, acc1b = _conv_call(g1a, g1b, src_blk, dst_blk)
    acc1 = jnp.concatenate([acc1a, acc1b], axis=1)

    g2 = _layer2_call(acc1, d0, d1, b1.reshape(1, -1), W2)
    z = jnp.zeros((NP, EMB), F32)
    p0, p1 = _conv_es_call(g2, z, src_blk, dst_blk)

    return _decoder_call(p0, p1, d0, d1, b2.reshape(1, -1), Wdec)
